# baseline (speedup 1.0000x reference)
import functools

import jax
import jax.numpy as jnp
from jax import lax
from jax.experimental import pallas as pl
from jax.experimental.pallas import tpu as pltpu
from jax.experimental.pallas import tpu_sc as plsc

N = 10000
E = 320000
D = 128
NC = 2
NS = 16
NW = NC * NS
EPW = E // NW
CHUNK = 125
NCHUNK = EPW // CHUNK
CH_S = 40
NCH_S = EPW // CH_S
NBUF = 1
MBUF = 2 * NBUF
AP = 624
ZW = 640
DW = 16

_mesh = plsc.VectorSubcoreMesh(core_axis_name="c", subcore_axis_name="s")


def _count_body(idxr, zrows, ones_hbm, out_hbm, idx_v, ones_v, acc, ssem):
    cid = lax.axis_index("c")
    sid = lax.axis_index("s")
    wid = cid * NS + sid

    pltpu.sync_copy(ones_hbm, ones_v)
    base = sid * AP
    pltpu.sync_copy(zrows, acc.at[pl.ds(base, ZW)])
    pltpu.sync_copy(idxr.at[wid], idx_v)
    plsc.subcore_barrier()

    @pl.loop(0, NCHUNK, step=8)
    def _(j0):
        for t in range(8):
            pltpu.async_copy(ones_v, acc.at[idx_v.at[j0 + t]], ssem, add=True)

        @pl.loop(0, 8)
        def _(_t):
            pltpu.make_async_copy(ones_v, acc.at[idx_v.at[j0]], ssem).wait()

    plsc.subcore_barrier()
    pltpu.sync_copy(acc.at[pl.ds(base, ZW)],
                    out_hbm.at[pl.ds(cid * N + base, ZW)])


_count = pl.kernel(
    _count_body,
    out_type=jax.ShapeDtypeStruct((NC * N, D), jnp.float32),
    mesh=_mesh,
    scratch_types=[
        pltpu.VMEM((NCHUNK, CHUNK), jnp.int32),
        pltpu.VMEM((CHUNK, D), jnp.float32),
        pltpu.VMEM_SHARED((N, D), jnp.float32),
        pltpu.SemaphoreType.DMA,
    ],
)


def _spmm_body(h_hbm, srcr4, dstr, zrows, out_hbm,
               src_r, dst_v, rows_v, acc, gsem, ssem, isem):
    cid = lax.axis_index("c")
    sid = lax.axis_index("s")
    wid = cid * NS + sid

    base = sid * AP
    pltpu.sync_copy(zrows, acc.at[pl.ds(base, ZW)])
    pltpu.sync_copy(dstr.at[wid], dst_v)
    pltpu.sync_copy(srcr4.at[wid, 0], src_r.at[0])
    plsc.subcore_barrier()

    pltpu.async_copy(h_hbm.at[src_r.at[0, 0]], rows_v.at[0], gsem.at[0])
    pltpu.async_copy(srcr4.at[wid, 1], src_r.at[1], isem.at[1])

    @pl.loop(0, NCH_S)
    def _(j):
        b = lax.rem(j, 2)
        bn = 1 - b
        pltpu.make_async_copy(h_hbm.at[src_r.at[b, 0]], rows_v.at[b],
                              gsem.at[b]).wait()
        pltpu.async_copy(rows_v.at[b], acc.at[dst_v.at[j]], ssem.at[b],
                         add=True)
        jn = j + 1

        @pl.when(jn < NCH_S)
        def _():
            pltpu.make_async_copy(srcr4.at[wid, 0], src_r.at[bn],
                                  isem.at[bn]).wait()

            @pl.when(jn >= 2)
            def _():
                pltpu.make_async_copy(rows_v.at[bn], acc.at[dst_v.at[j]],
                                      ssem.at[bn]).wait()

            pltpu.async_copy(h_hbm.at[src_r.at[bn, 0]], rows_v.at[bn],
                             gsem.at[bn])
            jn2 = j + 2

            @pl.when(jn2 < NCH_S)
            def _():
                pltpu.async_copy(srcr4.at[wid, jn2], src_r.at[b],
                                 isem.at[b])

    @pl.loop(0, 2)
    def _(b):
        pltpu.make_async_copy(rows_v.at[b], acc.at[dst_v.at[0]],
                              ssem.at[b]).wait()

    plsc.subcore_barrier()
    pltpu.sync_copy(acc.at[pl.ds(base, ZW)],
                    out_hbm.at[pl.ds(cid * N + base, ZW)])


_spmm = pl.kernel(
    _spmm_body,
    out_type=jax.ShapeDtypeStruct((NC * N, D), jnp.float32),
    mesh=_mesh,
    scratch_types=[
        pltpu.VMEM((2, 1, CH_S), jnp.int32),
        pltpu.VMEM((NCH_S, CH_S), jnp.int32),
        pltpu.VMEM((2, CH_S, D), jnp.float32),
        pltpu.VMEM_SHARED((N, D), jnp.float32),
        pltpu.SemaphoreType.DMA((2,)),
        pltpu.SemaphoreType.DMA((2,)),
        pltpu.SemaphoreType.DMA((2,)),
    ],
)


_BT = 1000


def _prep_body(x_ref, dop_ref, dip_ref, xs_ref, ns_ref, nd_ref):
    dout = dop_ref[0][:, :DW] + dop_ref[1][:, :DW]
    din = dip_ref[0][:, :DW] + dip_ref[1][:, :DW]
    ns = lax.rsqrt(jnp.maximum(dout, 1.0))
    nd = lax.rsqrt(jnp.maximum(din, 1.0))
    ns_ref[...] = ns
    nd_ref[...] = nd
    xs_ref[...] = x_ref[...] * ns[:, 0:1]


def _layer_body(relu_next, p_ref, ns_ref, nd_ref, w_ref, b_ref, o_ref):
    agg = (p_ref[0] + p_ref[1]) * nd_ref[...][:, 0:1]
    h = jnp.dot(agg, w_ref[...], preferred_element_type=jnp.float32)
    h = h + b_ref[...]
    if relu_next:
        h = jnp.maximum(h, 0.0) * ns_ref[...][:, 0:1]
    o_ref[...] = h


def _prep(x, dout_p, din_p):
    grid = N // _BT
    return pl.pallas_call(
        _prep_body,
        grid=(grid,),
        in_specs=[
            pl.BlockSpec((_BT, D), lambda i: (i, 0)),
            pl.BlockSpec((NC, _BT, D), lambda i: (0, i, 0)),
            pl.BlockSpec((NC, _BT, D), lambda i: (0, i, 0)),
        ],
        out_specs=[
            pl.BlockSpec((_BT, D), lambda i: (i, 0)),
            pl.BlockSpec((_BT, DW), lambda i: (i, 0)),
            pl.BlockSpec((_BT, DW), lambda i: (i, 0)),
        ],
        out_shape=[
            jax.ShapeDtypeStruct((N, D), jnp.float32),
            jax.ShapeDtypeStruct((N, DW), jnp.float32),
            jax.ShapeDtypeStruct((N, DW), jnp.float32),
        ],
    )(x, dout_p, din_p)


def _layer(parts, ns, nd, W, b, relu_next):
    grid = N // _BT
    return pl.pallas_call(
        functools.partial(_layer_body, relu_next),
        grid=(grid,),
        in_specs=[
            pl.BlockSpec((NC, _BT, D), lambda i: (0, i, 0)),
            pl.BlockSpec((_BT, DW), lambda i: (i, 0)),
            pl.BlockSpec((_BT, DW), lambda i: (i, 0)),
            pl.BlockSpec((D, D), lambda i: (0, 0)),
            pl.BlockSpec((1, D), lambda i: (0, 0)),
        ],
        out_specs=pl.BlockSpec((_BT, D), lambda i: (i, 0)),
        out_shape=jax.ShapeDtypeStruct((N, D), jnp.float32),
    )(parts, ns, nd, W, b.reshape(1, D))


def kernel(x, edge_index, W1, b1, W2, b2, W3, b3):
    ei = edge_index.astype(jnp.int32)
    srcr_c = ei[0].reshape(NW, NCHUNK, CHUNK)
    dstr_c = ei[1].reshape(NW, NCHUNK, CHUNK)
    srcr_s = ei[0].reshape(NW, NCH_S, 1, CH_S)
    dstr_s = ei[1].reshape(NW, NCH_S, CH_S)
    zrows = jnp.zeros((ZW, D), jnp.float32)
    ones = jnp.ones((CHUNK, D), jnp.float32)

    dout_p = _count(srcr_c, zrows, ones).reshape(NC, N, D)
    din_p = _count(dstr_c, zrows, ones).reshape(NC, N, D)
    h, ns, nd = _prep(x, dout_p, din_p)

    for W, b, relu_next in ((W1, b1, True), (W2, b2, True), (W3, b3, False)):
        parts = _spmm(h, srcr_s, dstr_s, zrows).reshape(NC, N, D)
        h = _layer(parts, ns, nd, W, b, relu_next)
    return h


# speedup vs baseline: 1.4882x; 1.4882x over previous
import functools

import jax
import jax.numpy as jnp
from jax import lax
from jax.experimental import pallas as pl
from jax.experimental.pallas import tpu as pltpu
from jax.experimental.pallas import tpu_sc as plsc

N = 10000
E = 320000
D = 128
NC = 2
NS = 16
NW = NC * NS
EPW = E // NW
CHUNK = 125
NCHUNK = EPW // CHUNK
CH_S = 50
NCH_S = EPW // CH_S
KB = 40
NBLK = NCH_S // KB
SLOTR = 2 * KB
MBUF = 4
NBUF = 2
AP = 624
ZW = 640
DW = 16

_mesh = plsc.VectorSubcoreMesh(core_axis_name="c", subcore_axis_name="s")


def _count_body(idxr, zrows, ones_hbm, out_hbm, idx_v, ones_v, acc, ssem):
    cid = lax.axis_index("c")
    sid = lax.axis_index("s")
    wid = cid * NS + sid

    pltpu.sync_copy(ones_hbm, ones_v)
    base = sid * AP
    pltpu.sync_copy(zrows, acc.at[pl.ds(base, ZW)])
    pltpu.sync_copy(idxr.at[wid], idx_v)
    plsc.subcore_barrier()

    @pl.loop(0, NCHUNK, step=8)
    def _(j0):
        for t in range(8):
            pltpu.async_copy(ones_v, acc.at[idx_v.at[j0 + t]], ssem, add=True)

        @pl.loop(0, 8)
        def _(_t):
            pltpu.make_async_copy(ones_v, acc.at[idx_v.at[j0]], ssem).wait()

    plsc.subcore_barrier()
    pltpu.sync_copy(acc.at[pl.ds(base, ZW)],
                    out_hbm.at[pl.ds(cid * N + base, ZW)])


_count = pl.kernel(
    _count_body,
    out_type=jax.ShapeDtypeStruct((NC * N, D), jnp.float32),
    mesh=_mesh,
    scratch_types=[
        pltpu.VMEM((NCHUNK, CHUNK), jnp.int32),
        pltpu.VMEM((CHUNK, D), jnp.float32),
        pltpu.VMEM_SHARED((N, D), jnp.float32),
        pltpu.SemaphoreType.DMA,
    ],
)


def _spmm_body(h_hbm, srcrB, dstrB, zrows, out_hbm,
               sidx, didx, rows_v, acc, gsem, ssem, bsem):
    cid = lax.axis_index("c")
    sid = lax.axis_index("s")
    wid = cid * NS + sid

    base = sid * AP
    pltpu.sync_copy(zrows, acc.at[pl.ds(base, ZW)])
    pltpu.sync_copy(srcrB.at[wid, 0], sidx.at[pl.ds(0, KB)])
    pltpu.sync_copy(dstrB.at[wid, 0], didx.at[pl.ds(0, KB)])
    pltpu.async_copy(srcrB.at[wid, 1], sidx.at[pl.ds(KB, KB)], bsem.at[1])
    pltpu.async_copy(dstrB.at[wid, 1], didx.at[pl.ds(KB, KB)], bsem.at[1])
    plsc.subcore_barrier()

    for b in range(NBUF):
        pltpu.async_copy(h_hbm.at[sidx.at[b]], rows_v.at[b], gsem.at[b])

    @pl.loop(0, NCH_S)
    def _(j):
        b = lax.rem(j, MBUF)
        pltpu.make_async_copy(h_hbm.at[sidx.at[0]], rows_v.at[b],
                              gsem.at[b]).wait()
        pltpu.async_copy(rows_v.at[b], acc.at[didx.at[lax.rem(j, SLOTR)]],
                         ssem.at[b], add=True)
        jn = j + NBUF

        @pl.when(jn < NCH_S)
        def _():
            bn = lax.rem(jn, MBUF)

            @pl.when(jn >= MBUF)
            def _():
                pltpu.make_async_copy(rows_v.at[bn], acc.at[didx.at[0]],
                                      ssem.at[bn]).wait()

            @pl.when(lax.rem(jn, KB) == 0)
            def _():
                p = lax.rem(jn // KB, 2)
                pltpu.make_async_copy(srcrB.at[wid, 0], sidx.at[pl.ds(0, KB)],
                                      bsem.at[p]).wait()
                pltpu.make_async_copy(dstrB.at[wid, 0], didx.at[pl.ds(0, KB)],
                                      bsem.at[p]).wait()

            pltpu.async_copy(h_hbm.at[sidx.at[lax.rem(jn, SLOTR)]],
                             rows_v.at[bn], gsem.at[bn])

        blk_next = j // KB + 1

        @pl.when(jnp.logical_and(lax.rem(j, KB) == 2,
                                 jnp.logical_and(blk_next >= 2,
                                                 blk_next < NBLK)))
        def _():
            p = lax.rem(blk_next, 2)
            off = pl.multiple_of(p * KB, 8)
            pltpu.async_copy(srcrB.at[wid, blk_next],
                             sidx.at[pl.ds(off, KB)], bsem.at[p])
            pltpu.async_copy(dstrB.at[wid, blk_next],
                             didx.at[pl.ds(off, KB)], bsem.at[p])

    @pl.loop(0, MBUF)
    def _(b):
        pltpu.make_async_copy(rows_v.at[b], acc.at[didx.at[0]],
                              ssem.at[b]).wait()

    plsc.subcore_barrier()
    pltpu.sync_copy(acc.at[pl.ds(base, ZW)],
                    out_hbm.at[pl.ds(cid * N + base, ZW)])


_spmm = pl.kernel(
    _spmm_body,
    out_type=jax.ShapeDtypeStruct((NC * N, D), jnp.float32),
    mesh=_mesh,
    scratch_types=[
        pltpu.VMEM((SLOTR, CH_S), jnp.int32),
        pltpu.VMEM((SLOTR, CH_S), jnp.int32),
        pltpu.VMEM((MBUF, CH_S, D), jnp.float32),
        pltpu.VMEM_SHARED((N, D), jnp.float32),
        pltpu.SemaphoreType.DMA((MBUF,)),
        pltpu.SemaphoreType.DMA((MBUF,)),
        pltpu.SemaphoreType.DMA((2,)),
    ],
)


_BT = 1000


def _prep_body(x_ref, dop_ref, dip_ref, xs_ref, ns_ref, nd_ref):
    dout = dop_ref[0][:, :DW] + dop_ref[1][:, :DW]
    din = dip_ref[0][:, :DW] + dip_ref[1][:, :DW]
    ns = lax.rsqrt(jnp.maximum(dout, 1.0))
    nd = lax.rsqrt(jnp.maximum(din, 1.0))
    ns_ref[...] = ns
    nd_ref[...] = nd
    xs_ref[...] = x_ref[...] * ns[:, 0:1]


def _layer_body(relu_next, p_ref, ns_ref, nd_ref, w_ref, b_ref, o_ref):
    agg = (p_ref[0] + p_ref[1]) * nd_ref[...][:, 0:1]
    h = jnp.dot(agg, w_ref[...], preferred_element_type=jnp.float32)
    h = h + b_ref[...]
    if relu_next:
        h = jnp.maximum(h, 0.0) * ns_ref[...][:, 0:1]
    o_ref[...] = h


def _prep(x, dout_p, din_p):
    grid = N // _BT
    return pl.pallas_call(
        _prep_body,
        grid=(grid,),
        in_specs=[
            pl.BlockSpec((_BT, D), lambda i: (i, 0)),
            pl.BlockSpec((NC, _BT, D), lambda i: (0, i, 0)),
            pl.BlockSpec((NC, _BT, D), lambda i: (0, i, 0)),
        ],
        out_specs=[
            pl.BlockSpec((_BT, D), lambda i: (i, 0)),
            pl.BlockSpec((_BT, DW), lambda i: (i, 0)),
            pl.BlockSpec((_BT, DW), lambda i: (i, 0)),
        ],
        out_shape=[
            jax.ShapeDtypeStruct((N, D), jnp.float32),
            jax.ShapeDtypeStruct((N, DW), jnp.float32),
            jax.ShapeDtypeStruct((N, DW), jnp.float32),
        ],
    )(x, dout_p, din_p)


def _layer(parts, ns, nd, W, b, relu_next):
    grid = N // _BT
    return pl.pallas_call(
        functools.partial(_layer_body, relu_next),
        grid=(grid,),
        in_specs=[
            pl.BlockSpec((NC, _BT, D), lambda i: (0, i, 0)),
            pl.BlockSpec((_BT, DW), lambda i: (i, 0)),
            pl.BlockSpec((_BT, DW), lambda i: (i, 0)),
            pl.BlockSpec((D, D), lambda i: (0, 0)),
            pl.BlockSpec((1, D), lambda i: (0, 0)),
        ],
        out_specs=pl.BlockSpec((_BT, D), lambda i: (i, 0)),
        out_shape=jax.ShapeDtypeStruct((N, D), jnp.float32),
    )(parts, ns, nd, W, b.reshape(1, D))


def kernel(x, edge_index, W1, b1, W2, b2, W3, b3):
    ei = edge_index.astype(jnp.int32)
    srcr_c = ei[0].reshape(NW, NCHUNK, CHUNK)
    dstr_c = ei[1].reshape(NW, NCHUNK, CHUNK)
    srcr_s = ei[0].reshape(NW, NBLK, KB, CH_S)
    dstr_s = ei[1].reshape(NW, NBLK, KB, CH_S)
    zrows = jnp.zeros((ZW, D), jnp.float32)
    ones = jnp.ones((CHUNK, D), jnp.float32)

    dout_p = _count(srcr_c, zrows, ones).reshape(NC, N, D)
    din_p = _count(dstr_c, zrows, ones).reshape(NC, N, D)
    h, ns, nd = _prep(x, dout_p, din_p)

    for W, b, relu_next in ((W1, b1, True), (W2, b2, True), (W3, b3, False)):
        parts = _spmm(h, srcr_s, dstr_s, zrows).reshape(NC, N, D)
        h = _layer(parts, ns, nd, W, b, relu_next)
    return h


# speedup vs baseline: 1.6107x; 1.0823x over previous
import functools

import jax
import jax.numpy as jnp
from jax import lax
from jax.experimental import pallas as pl
from jax.experimental.pallas import tpu as pltpu
from jax.experimental.pallas import tpu_sc as plsc

N = 10000
E = 320000
D = 128
NC = 2
NS = 16
NW = NC * NS
EPW = E // NW
CHUNK = 125
NCHUNK = EPW // CHUNK
CH_S = 50
NCH_S = EPW // CH_S
KB = 8
NBLK = NCH_S // KB
SLOTR = 2 * KB
MBUF = 6
NBUF = 3
AP = 624
ZW = 640
DW = 16

_mesh = plsc.VectorSubcoreMesh(core_axis_name="c", subcore_axis_name="s")


def _count_body(idxr, zrows, ones_hbm, out_hbm, idx_v, ones_v, acc, ssem):
    cid = lax.axis_index("c")
    sid = lax.axis_index("s")
    wid = cid * NS + sid

    pltpu.sync_copy(ones_hbm, ones_v)
    base = sid * AP
    pltpu.sync_copy(zrows, acc.at[pl.ds(base, ZW)])
    pltpu.sync_copy(idxr.at[wid], idx_v)
    plsc.subcore_barrier()

    @pl.loop(0, NCHUNK, step=8)
    def _(j0):
        for t in range(8):
            pltpu.async_copy(ones_v, acc.at[idx_v.at[j0 + t]], ssem, add=True)

        @pl.loop(0, 8)
        def _(_t):
            pltpu.make_async_copy(ones_v, acc.at[idx_v.at[j0]], ssem).wait()

    plsc.subcore_barrier()
    pltpu.sync_copy(acc.at[pl.ds(base, ZW)],
                    out_hbm.at[pl.ds(cid * N + base, ZW)])


_count = pl.kernel(
    _count_body,
    out_type=jax.ShapeDtypeStruct((NC * N, D), jnp.float32),
    mesh=_mesh,
    scratch_types=[
        pltpu.VMEM((NCHUNK, CHUNK), jnp.int32),
        pltpu.VMEM((CHUNK, D), jnp.float32),
        pltpu.VMEM_SHARED((N, D), jnp.float32),
        pltpu.SemaphoreType.DMA,
    ],
)


def _spmm_body(h_hbm, srcrB, dstrB, zrows, out_hbm,
               sidx, didx, rows_v, acc, gsem, ssem, bsem):
    cid = lax.axis_index("c")
    sid = lax.axis_index("s")
    wid = cid * NS + sid

    base = sid * AP
    pltpu.sync_copy(zrows, acc.at[pl.ds(base, ZW)])
    pltpu.sync_copy(srcrB.at[wid, 0], sidx.at[pl.ds(0, KB)])
    pltpu.sync_copy(dstrB.at[wid, 0], didx.at[pl.ds(0, KB)])
    pltpu.async_copy(srcrB.at[wid, 1], sidx.at[pl.ds(KB, KB)], bsem.at[1])
    pltpu.async_copy(dstrB.at[wid, 1], didx.at[pl.ds(KB, KB)], bsem.at[1])
    plsc.subcore_barrier()

    for b in range(NBUF):
        pltpu.async_copy(h_hbm.at[sidx.at[b]], rows_v.at[b], gsem.at[b])

    @pl.loop(0, NCH_S)
    def _(j):
        b = lax.rem(j, MBUF)
        pltpu.make_async_copy(h_hbm.at[sidx.at[0]], rows_v.at[b],
                              gsem.at[b]).wait()
        pltpu.async_copy(rows_v.at[b], acc.at[didx.at[lax.rem(j, SLOTR)]],
                         ssem.at[b], add=True)
        jn = j + NBUF

        @pl.when(jn < NCH_S)
        def _():
            bn = lax.rem(jn, MBUF)

            @pl.when(jn >= MBUF)
            def _():
                pltpu.make_async_copy(rows_v.at[bn], acc.at[didx.at[0]],
                                      ssem.at[bn]).wait()

            @pl.when(lax.rem(jn, KB) == 0)
            def _():
                p = lax.rem(jn // KB, 2)
                pltpu.make_async_copy(srcrB.at[wid, 0], sidx.at[pl.ds(0, KB)],
                                      bsem.at[p]).wait()
                pltpu.make_async_copy(dstrB.at[wid, 0], didx.at[pl.ds(0, KB)],
                                      bsem.at[p]).wait()

            pltpu.async_copy(h_hbm.at[sidx.at[lax.rem(jn, SLOTR)]],
                             rows_v.at[bn], gsem.at[bn])

        blk_next = j // KB + 1

        @pl.when(jnp.logical_and(lax.rem(j, KB) == 2,
                                 jnp.logical_and(blk_next >= 2,
                                                 blk_next < NBLK)))
        def _():
            p = lax.rem(blk_next, 2)
            off = pl.multiple_of(p * KB, 8)
            pltpu.async_copy(srcrB.at[wid, blk_next],
                             sidx.at[pl.ds(off, KB)], bsem.at[p])
            pltpu.async_copy(dstrB.at[wid, blk_next],
                             didx.at[pl.ds(off, KB)], bsem.at[p])

    @pl.loop(0, MBUF)
    def _(b):
        pltpu.make_async_copy(rows_v.at[b], acc.at[didx.at[0]],
                              ssem.at[b]).wait()

    plsc.subcore_barrier()
    pltpu.sync_copy(acc.at[pl.ds(base, ZW)],
                    out_hbm.at[pl.ds(cid * N + base, ZW)])


_spmm = pl.kernel(
    _spmm_body,
    out_type=jax.ShapeDtypeStruct((NC * N, D), jnp.float32),
    mesh=_mesh,
    scratch_types=[
        pltpu.VMEM((SLOTR, CH_S), jnp.int32),
        pltpu.VMEM((SLOTR, CH_S), jnp.int32),
        pltpu.VMEM((MBUF, CH_S, D), jnp.float32),
        pltpu.VMEM_SHARED((N, D), jnp.float32),
        pltpu.SemaphoreType.DMA((MBUF,)),
        pltpu.SemaphoreType.DMA((MBUF,)),
        pltpu.SemaphoreType.DMA((2,)),
    ],
)


_BT = 1000


def _prep_body(x_ref, dop_ref, dip_ref, xs_ref, ns_ref, nd_ref):
    dout = dop_ref[0][:, :DW] + dop_ref[1][:, :DW]
    din = dip_ref[0][:, :DW] + dip_ref[1][:, :DW]
    ns = lax.rsqrt(jnp.maximum(dout, 1.0))
    nd = lax.rsqrt(jnp.maximum(din, 1.0))
    ns_ref[...] = ns
    nd_ref[...] = nd
    xs_ref[...] = x_ref[...] * ns[:, 0:1]


def _layer_body(relu_next, p_ref, ns_ref, nd_ref, w_ref, b_ref, o_ref):
    agg = (p_ref[0] + p_ref[1]) * nd_ref[...][:, 0:1]
    h = jnp.dot(agg, w_ref[...], preferred_element_type=jnp.float32)
    h = h + b_ref[...]
    if relu_next:
        h = jnp.maximum(h, 0.0) * ns_ref[...][:, 0:1]
    o_ref[...] = h


def _prep(x, dout_p, din_p):
    grid = N // _BT
    return pl.pallas_call(
        _prep_body,
        grid=(grid,),
        in_specs=[
            pl.BlockSpec((_BT, D), lambda i: (i, 0)),
            pl.BlockSpec((NC, _BT, D), lambda i: (0, i, 0)),
            pl.BlockSpec((NC, _BT, D), lambda i: (0, i, 0)),
        ],
        out_specs=[
            pl.BlockSpec((_BT, D), lambda i: (i, 0)),
            pl.BlockSpec((_BT, DW), lambda i: (i, 0)),
            pl.BlockSpec((_BT, DW), lambda i: (i, 0)),
        ],
        out_shape=[
            jax.ShapeDtypeStruct((N, D), jnp.float32),
            jax.ShapeDtypeStruct((N, DW), jnp.float32),
            jax.ShapeDtypeStruct((N, DW), jnp.float32),
        ],
    )(x, dout_p, din_p)


def _layer(parts, ns, nd, W, b, relu_next):
    grid = N // _BT
    return pl.pallas_call(
        functools.partial(_layer_body, relu_next),
        grid=(grid,),
        in_specs=[
            pl.BlockSpec((NC, _BT, D), lambda i: (0, i, 0)),
            pl.BlockSpec((_BT, DW), lambda i: (i, 0)),
            pl.BlockSpec((_BT, DW), lambda i: (i, 0)),
            pl.BlockSpec((D, D), lambda i: (0, 0)),
            pl.BlockSpec((1, D), lambda i: (0, 0)),
        ],
        out_specs=pl.BlockSpec((_BT, D), lambda i: (i, 0)),
        out_shape=jax.ShapeDtypeStruct((N, D), jnp.float32),
    )(parts, ns, nd, W, b.reshape(1, D))


def kernel(x, edge_index, W1, b1, W2, b2, W3, b3):
    ei = edge_index.astype(jnp.int32)
    srcr_c = ei[0].reshape(NW, NCHUNK, CHUNK)
    dstr_c = ei[1].reshape(NW, NCHUNK, CHUNK)
    srcr_s = ei[0].reshape(NW, NBLK, KB, CH_S)
    dstr_s = ei[1].reshape(NW, NBLK, KB, CH_S)
    zrows = jnp.zeros((ZW, D), jnp.float32)
    ones = jnp.ones((CHUNK, D), jnp.float32)

    dout_p = _count(srcr_c, zrows, ones).reshape(NC, N, D)
    din_p = _count(dstr_c, zrows, ones).reshape(NC, N, D)
    h, ns, nd = _prep(x, dout_p, din_p)

    for W, b, relu_next in ((W1, b1, True), (W2, b2, True), (W3, b3, False)):
        parts = _spmm(h, srcr_s, dstr_s, zrows).reshape(NC, N, D)
        h = _layer(parts, ns, nd, W, b, relu_next)
    return h


# speedup vs baseline: 1.6464x; 1.0222x over previous
import functools

import jax
import jax.numpy as jnp
from jax import lax
from jax.experimental import pallas as pl
from jax.experimental.pallas import tpu as pltpu
from jax.experimental.pallas import tpu_sc as plsc

N = 10000
E = 320000
D = 128
NC = 2
NS = 16
NW = NC * NS
EPW = E // NW
CHUNK = 125
NCHUNK = EPW // CHUNK
CH_S = 50
NCH_S = EPW // CH_S
KB = 8
NBLK = NCH_S // KB
SLOTR = 2 * KB
MBUF = 6
NBUF = 3
AP = 624
ZW = 640
DW = 16

_mesh = plsc.VectorSubcoreMesh(core_axis_name="c", subcore_axis_name="s")


def _count_body(srcr, dstr, zrows, ones_hbm, dout_hbm, dsum_hbm,
                sidx_v, didx_v, ones_v, acc, ssem):
    cid = lax.axis_index("c")
    sid = lax.axis_index("s")
    wid = cid * NS + sid

    pltpu.sync_copy(ones_hbm, ones_v)
    base = sid * AP
    pltpu.sync_copy(zrows, acc.at[pl.ds(base, ZW)])
    pltpu.sync_copy(srcr.at[wid], sidx_v)
    pltpu.sync_copy(dstr.at[wid], didx_v)
    plsc.subcore_barrier()

    @pl.loop(0, NCHUNK, step=8)
    def _(j0):
        for t in range(8):
            pltpu.async_copy(ones_v, acc.at[sidx_v.at[j0 + t]], ssem, add=True)

        @pl.loop(0, 8)
        def _(_t):
            pltpu.make_async_copy(ones_v, acc.at[sidx_v.at[j0]], ssem).wait()

    plsc.subcore_barrier()
    pltpu.sync_copy(acc.at[pl.ds(base, ZW)],
                    dout_hbm.at[pl.ds(cid * N + base, ZW)])
    plsc.subcore_barrier()

    @pl.loop(0, NCHUNK, step=8)
    def _(j0):
        for t in range(8):
            pltpu.async_copy(ones_v, acc.at[didx_v.at[j0 + t]], ssem, add=True)

        @pl.loop(0, 8)
        def _(_t):
            pltpu.make_async_copy(ones_v, acc.at[didx_v.at[j0]], ssem).wait()

    plsc.subcore_barrier()
    pltpu.sync_copy(acc.at[pl.ds(base, ZW)],
                    dsum_hbm.at[pl.ds(cid * N + base, ZW)])


_count = pl.kernel(
    _count_body,
    out_type=[
        jax.ShapeDtypeStruct((NC * N, D), jnp.float32),
        jax.ShapeDtypeStruct((NC * N, D), jnp.float32),
    ],
    mesh=_mesh,
    scratch_types=[
        pltpu.VMEM((NCHUNK, CHUNK), jnp.int32),
        pltpu.VMEM((NCHUNK, CHUNK), jnp.int32),
        pltpu.VMEM((CHUNK, D), jnp.float32),
        pltpu.VMEM_SHARED((N, D), jnp.float32),
        pltpu.SemaphoreType.DMA,
    ],
)


def _spmm_body(h_hbm, srcrB, dstrB, zrows, out_hbm,
               sidx, didx, rows_v, acc, gsem, ssem, bsem):
    cid = lax.axis_index("c")
    sid = lax.axis_index("s")
    wid = cid * NS + sid

    base = sid * AP
    pltpu.sync_copy(zrows, acc.at[pl.ds(base, ZW)])
    pltpu.sync_copy(srcrB.at[wid, 0], sidx.at[pl.ds(0, KB)])
    pltpu.sync_copy(dstrB.at[wid, 0], didx.at[pl.ds(0, KB)])
    pltpu.async_copy(srcrB.at[wid, 1], sidx.at[pl.ds(KB, KB)], bsem.at[1])
    pltpu.async_copy(dstrB.at[wid, 1], didx.at[pl.ds(KB, KB)], bsem.at[1])
    plsc.subcore_barrier()

    for b in range(NBUF):
        pltpu.async_copy(h_hbm.at[sidx.at[b]], rows_v.at[b], gsem.at[b])

    @pl.loop(0, NCH_S)
    def _(j):
        b = lax.rem(j, MBUF)
        pltpu.make_async_copy(h_hbm.at[sidx.at[0]], rows_v.at[b],
                              gsem.at[b]).wait()
        pltpu.async_copy(rows_v.at[b], acc.at[didx.at[lax.rem(j, SLOTR)]],
                         ssem.at[b], add=True)
        jn = j + NBUF

        @pl.when(jn < NCH_S)
        def _():
            bn = lax.rem(jn, MBUF)

            @pl.when(jn >= MBUF)
            def _():
                pltpu.make_async_copy(rows_v.at[bn], acc.at[didx.at[0]],
                                      ssem.at[bn]).wait()

            @pl.when(lax.rem(jn, KB) == 0)
            def _():
                p = lax.rem(jn // KB, 2)
                pltpu.make_async_copy(srcrB.at[wid, 0], sidx.at[pl.ds(0, KB)],
                                      bsem.at[p]).wait()
                pltpu.make_async_copy(dstrB.at[wid, 0], didx.at[pl.ds(0, KB)],
                                      bsem.at[p]).wait()

            pltpu.async_copy(h_hbm.at[sidx.at[lax.rem(jn, SLOTR)]],
                             rows_v.at[bn], gsem.at[bn])

        blk_next = j // KB + 1

        @pl.when(jnp.logical_and(lax.rem(j, KB) == 2,
                                 jnp.logical_and(blk_next >= 2,
                                                 blk_next < NBLK)))
        def _():
            p = lax.rem(blk_next, 2)
            off = pl.multiple_of(p * KB, 8)
            pltpu.async_copy(srcrB.at[wid, blk_next],
                             sidx.at[pl.ds(off, KB)], bsem.at[p])
            pltpu.async_copy(dstrB.at[wid, blk_next],
                             didx.at[pl.ds(off, KB)], bsem.at[p])

    @pl.loop(0, MBUF)
    def _(b):
        pltpu.make_async_copy(rows_v.at[b], acc.at[didx.at[0]],
                              ssem.at[b]).wait()

    plsc.subcore_barrier()
    pltpu.sync_copy(acc.at[pl.ds(base, ZW)],
                    out_hbm.at[pl.ds(cid * N + base, ZW)])


_spmm = pl.kernel(
    _spmm_body,
    out_type=jax.ShapeDtypeStruct((NC * N, D), jnp.float32),
    mesh=_mesh,
    scratch_types=[
        pltpu.VMEM((SLOTR, CH_S), jnp.int32),
        pltpu.VMEM((SLOTR, CH_S), jnp.int32),
        pltpu.VMEM((MBUF, CH_S, D), jnp.float32),
        pltpu.VMEM_SHARED((N, D), jnp.float32),
        pltpu.SemaphoreType.DMA((MBUF,)),
        pltpu.SemaphoreType.DMA((MBUF,)),
        pltpu.SemaphoreType.DMA((2,)),
    ],
)


_BT = 1000


def _prep_body(x_ref, dop_ref, dip_ref, xs_ref, ns_ref, nd_ref):
    dout = dop_ref[0][:, :DW] + dop_ref[1][:, :DW]
    din = dip_ref[0][:, :DW] + dip_ref[1][:, :DW] - dout
    ns = lax.rsqrt(jnp.maximum(dout, 1.0))
    nd = lax.rsqrt(jnp.maximum(din, 1.0))
    ns_ref[...] = ns
    nd_ref[...] = nd
    xs_ref[...] = x_ref[...] * ns[:, 0:1]


def _layer_body(relu_next, p_ref, ns_ref, nd_ref, w_ref, b_ref, o_ref):
    agg = (p_ref[0] + p_ref[1]) * nd_ref[...][:, 0:1]
    h = jnp.dot(agg, w_ref[...], preferred_element_type=jnp.float32)
    h = h + b_ref[...]
    if relu_next:
        h = jnp.maximum(h, 0.0) * ns_ref[...][:, 0:1]
    o_ref[...] = h


def _prep(x, dout_p, din_p):
    grid = N // _BT
    return pl.pallas_call(
        _prep_body,
        grid=(grid,),
        in_specs=[
            pl.BlockSpec((_BT, D), lambda i: (i, 0)),
            pl.BlockSpec((NC, _BT, D), lambda i: (0, i, 0)),
            pl.BlockSpec((NC, _BT, D), lambda i: (0, i, 0)),
        ],
        out_specs=[
            pl.BlockSpec((_BT, D), lambda i: (i, 0)),
            pl.BlockSpec((_BT, DW), lambda i: (i, 0)),
            pl.BlockSpec((_BT, DW), lambda i: (i, 0)),
        ],
        out_shape=[
            jax.ShapeDtypeStruct((N, D), jnp.float32),
            jax.ShapeDtypeStruct((N, DW), jnp.float32),
            jax.ShapeDtypeStruct((N, DW), jnp.float32),
        ],
    )(x, dout_p, din_p)


def _layer(parts, ns, nd, W, b, relu_next):
    grid = N // _BT
    return pl.pallas_call(
        functools.partial(_layer_body, relu_next),
        grid=(grid,),
        in_specs=[
            pl.BlockSpec((NC, _BT, D), lambda i: (0, i, 0)),
            pl.BlockSpec((_BT, DW), lambda i: (i, 0)),
            pl.BlockSpec((_BT, DW), lambda i: (i, 0)),
            pl.BlockSpec((D, D), lambda i: (0, 0)),
            pl.BlockSpec((1, D), lambda i: (0, 0)),
        ],
        out_specs=pl.BlockSpec((_BT, D), lambda i: (i, 0)),
        out_shape=jax.ShapeDtypeStruct((N, D), jnp.float32),
    )(parts, ns, nd, W, b.reshape(1, D))


def kernel(x, edge_index, W1, b1, W2, b2, W3, b3):
    ei = edge_index.astype(jnp.int32)
    srcr_c = ei[0].reshape(NW, NCHUNK, CHUNK)
    dstr_c = ei[1].reshape(NW, NCHUNK, CHUNK)
    srcr_s = ei[0].reshape(NW, NBLK, KB, CH_S)
    dstr_s = ei[1].reshape(NW, NBLK, KB, CH_S)
    zrows = jnp.zeros((ZW, D), jnp.float32)
    ones = jnp.ones((CHUNK, D), jnp.float32)

    dout_p, dsum_p = _count(srcr_c, dstr_c, zrows, ones)
    dout_p = dout_p.reshape(NC, N, D)
    din_p = dsum_p.reshape(NC, N, D)
    h, ns, nd = _prep(x, dout_p, din_p)

    for W, b, relu_next in ((W1, b1, True), (W2, b2, True), (W3, b3, False)):
        parts = _spmm(h, srcr_s, dstr_s, zrows).reshape(NC, N, D)
        h = _layer(parts, ns, nd, W, b, relu_next)
    return h


# speedup vs baseline: 1.7406x; 1.0572x over previous
import functools

import jax
import jax.numpy as jnp
from jax import lax
from jax.experimental import pallas as pl
from jax.experimental.pallas import tpu as pltpu
from jax.experimental.pallas import tpu_sc as plsc

N = 10000
E = 320000
D = 128
NC = 2
NS = 16
NW = NC * NS
EPW = E // NW
CHUNK = 125
NCHUNK = EPW // CHUNK
CH_S = 50
NCH_S = EPW // CH_S
KB = 8
NBLK = NCH_S // KB
SLOTR = 2 * KB
MBUF = 6
NBUF = 4
AP = 624
ZW = 640
DW = 16

_mesh = plsc.VectorSubcoreMesh(core_axis_name="c", subcore_axis_name="s")


def _count_body(srcr, dstr, zrows, ones_hbm, dout_hbm, dsum_hbm,
                sidx_v, didx_v, ones_v, acc, ssem):
    cid = lax.axis_index("c")
    sid = lax.axis_index("s")
    wid = cid * NS + sid

    pltpu.sync_copy(ones_hbm, ones_v)
    base = sid * AP
    pltpu.sync_copy(zrows, acc.at[pl.ds(base, ZW)])
    pltpu.sync_copy(srcr.at[wid], sidx_v)
    pltpu.sync_copy(dstr.at[wid], didx_v)
    plsc.subcore_barrier()

    @pl.loop(0, NCHUNK, step=8)
    def _(j0):
        for t in range(8):
            pltpu.async_copy(ones_v, acc.at[sidx_v.at[j0 + t]], ssem, add=True)

        @pl.loop(0, 8)
        def _(_t):
            pltpu.make_async_copy(ones_v, acc.at[sidx_v.at[j0]], ssem).wait()

    plsc.subcore_barrier()
    pltpu.sync_copy(acc.at[pl.ds(base, ZW)],
                    dout_hbm.at[pl.ds(cid * N + base, ZW)])
    plsc.subcore_barrier()

    @pl.loop(0, NCHUNK, step=8)
    def _(j0):
        for t in range(8):
            pltpu.async_copy(ones_v, acc.at[didx_v.at[j0 + t]], ssem, add=True)

        @pl.loop(0, 8)
        def _(_t):
            pltpu.make_async_copy(ones_v, acc.at[didx_v.at[j0]], ssem).wait()

    plsc.subcore_barrier()
    pltpu.sync_copy(acc.at[pl.ds(base, ZW)],
                    dsum_hbm.at[pl.ds(cid * N + base, ZW)])


_count = pl.kernel(
    _count_body,
    out_type=[
        jax.ShapeDtypeStruct((NC * N, D), jnp.float32),
        jax.ShapeDtypeStruct((NC * N, D), jnp.float32),
    ],
    mesh=_mesh,
    scratch_types=[
        pltpu.VMEM((NCHUNK, CHUNK), jnp.int32),
        pltpu.VMEM((NCHUNK, CHUNK), jnp.int32),
        pltpu.VMEM((CHUNK, D), jnp.float32),
        pltpu.VMEM_SHARED((N, D), jnp.float32),
        pltpu.SemaphoreType.DMA,
    ],
)


def _spmm_body(h_hbm, srcrB, dstrB, zrows, out_hbm,
               sidx, didx, rows_v, acc, gsem, ssem, bsem):
    cid = lax.axis_index("c")
    sid = lax.axis_index("s")
    wid = cid * NS + sid

    base = sid * AP
    pltpu.sync_copy(zrows, acc.at[pl.ds(base, ZW)])
    pltpu.sync_copy(srcrB.at[wid, 0], sidx.at[pl.ds(0, KB)])
    pltpu.sync_copy(dstrB.at[wid, 0], didx.at[pl.ds(0, KB)])
    pltpu.async_copy(srcrB.at[wid, 1], sidx.at[pl.ds(KB, KB)], bsem.at[1])
    pltpu.async_copy(dstrB.at[wid, 1], didx.at[pl.ds(KB, KB)], bsem.at[1])
    plsc.subcore_barrier()

    for b in range(NBUF):
        pltpu.async_copy(h_hbm.at[sidx.at[b]], rows_v.at[b], gsem.at[b])

    @pl.loop(0, NCH_S)
    def _(j):
        b = lax.rem(j, MBUF)
        pltpu.make_async_copy(h_hbm.at[sidx.at[0]], rows_v.at[b],
                              gsem.at[b]).wait()
        pltpu.async_copy(rows_v.at[b], acc.at[didx.at[lax.rem(j, SLOTR)]],
                         ssem.at[b], add=True)
        jn = j + NBUF

        @pl.when(jn < NCH_S)
        def _():
            bn = lax.rem(jn, MBUF)

            @pl.when(jn >= MBUF)
            def _():
                pltpu.make_async_copy(rows_v.at[bn], acc.at[didx.at[0]],
                                      ssem.at[bn]).wait()

            @pl.when(lax.rem(jn, KB) == 0)
            def _():
                p = lax.rem(jn // KB, 2)
                pltpu.make_async_copy(srcrB.at[wid, 0], sidx.at[pl.ds(0, KB)],
                                      bsem.at[p]).wait()
                pltpu.make_async_copy(dstrB.at[wid, 0], didx.at[pl.ds(0, KB)],
                                      bsem.at[p]).wait()

            pltpu.async_copy(h_hbm.at[sidx.at[lax.rem(jn, SLOTR)]],
                             rows_v.at[bn], gsem.at[bn])

        blk_next = j // KB + 1

        @pl.when(jnp.logical_and(lax.rem(j, KB) == 2,
                                 jnp.logical_and(blk_next >= 2,
                                                 blk_next < NBLK)))
        def _():
            p = lax.rem(blk_next, 2)
            off = pl.multiple_of(p * KB, 8)
            pltpu.async_copy(srcrB.at[wid, blk_next],
                             sidx.at[pl.ds(off, KB)], bsem.at[p])
            pltpu.async_copy(dstrB.at[wid, blk_next],
                             didx.at[pl.ds(off, KB)], bsem.at[p])

    @pl.loop(0, MBUF)
    def _(b):
        pltpu.make_async_copy(rows_v.at[b], acc.at[didx.at[0]],
                              ssem.at[b]).wait()

    plsc.subcore_barrier()
    pltpu.sync_copy(acc.at[pl.ds(base, ZW)],
                    out_hbm.at[pl.ds(cid * N + base, ZW)])


_spmm = pl.kernel(
    _spmm_body,
    out_type=jax.ShapeDtypeStruct((NC * N, D), jnp.float32),
    mesh=_mesh,
    scratch_types=[
        pltpu.VMEM((SLOTR, CH_S), jnp.int32),
        pltpu.VMEM((SLOTR, CH_S), jnp.int32),
        pltpu.VMEM((MBUF, CH_S, D), jnp.float32),
        pltpu.VMEM_SHARED((N, D), jnp.float32),
        pltpu.SemaphoreType.DMA((MBUF,)),
        pltpu.SemaphoreType.DMA((MBUF,)),
        pltpu.SemaphoreType.DMA((2,)),
    ],
)


_BT = 1000


def _prep_body(x_ref, dop_ref, dip_ref, xs_ref, ns_ref, nd_ref):
    dout = dop_ref[0][:, :DW] + dop_ref[1][:, :DW]
    din = dip_ref[0][:, :DW] + dip_ref[1][:, :DW] - dout
    ns = lax.rsqrt(jnp.maximum(dout, 1.0))
    nd = lax.rsqrt(jnp.maximum(din, 1.0))
    ns_ref[...] = ns
    nd_ref[...] = nd
    xs_ref[...] = x_ref[...] * ns[:, 0:1]


def _layer_body(relu_next, p_ref, ns_ref, nd_ref, w_ref, b_ref, o_ref):
    agg = (p_ref[0] + p_ref[1]) * nd_ref[...][:, 0:1]
    h = jnp.dot(agg, w_ref[...], preferred_element_type=jnp.float32)
    h = h + b_ref[...]
    if relu_next:
        h = jnp.maximum(h, 0.0) * ns_ref[...][:, 0:1]
    o_ref[...] = h


def _prep(x, dout_p, din_p):
    grid = N // _BT
    return pl.pallas_call(
        _prep_body,
        grid=(grid,),
        in_specs=[
            pl.BlockSpec((_BT, D), lambda i: (i, 0)),
            pl.BlockSpec((NC, _BT, D), lambda i: (0, i, 0)),
            pl.BlockSpec((NC, _BT, D), lambda i: (0, i, 0)),
        ],
        out_specs=[
            pl.BlockSpec((_BT, D), lambda i: (i, 0)),
            pl.BlockSpec((_BT, DW), lambda i: (i, 0)),
            pl.BlockSpec((_BT, DW), lambda i: (i, 0)),
        ],
        out_shape=[
            jax.ShapeDtypeStruct((N, D), jnp.float32),
            jax.ShapeDtypeStruct((N, DW), jnp.float32),
            jax.ShapeDtypeStruct((N, DW), jnp.float32),
        ],
    )(x, dout_p, din_p)


def _layer(parts, ns, nd, W, b, relu_next):
    grid = N // _BT
    return pl.pallas_call(
        functools.partial(_layer_body, relu_next),
        grid=(grid,),
        in_specs=[
            pl.BlockSpec((NC, _BT, D), lambda i: (0, i, 0)),
            pl.BlockSpec((_BT, DW), lambda i: (i, 0)),
            pl.BlockSpec((_BT, DW), lambda i: (i, 0)),
            pl.BlockSpec((D, D), lambda i: (0, 0)),
            pl.BlockSpec((1, D), lambda i: (0, 0)),
        ],
        out_specs=pl.BlockSpec((_BT, D), lambda i: (i, 0)),
        out_shape=jax.ShapeDtypeStruct((N, D), jnp.float32),
    )(parts, ns, nd, W, b.reshape(1, D))


def kernel(x, edge_index, W1, b1, W2, b2, W3, b3):
    ei = edge_index.astype(jnp.int32)
    srcr_c = ei[0].reshape(NW, NCHUNK, CHUNK)
    dstr_c = ei[1].reshape(NW, NCHUNK, CHUNK)
    srcr_s = ei[0].reshape(NW, NBLK, KB, CH_S)
    dstr_s = ei[1].reshape(NW, NBLK, KB, CH_S)
    zrows = jnp.zeros((ZW, D), jnp.float32)
    ones = jnp.ones((CHUNK, D), jnp.float32)

    dout_p, dsum_p = _count(srcr_c, dstr_c, zrows, ones)
    dout_p = dout_p.reshape(NC, N, D)
    din_p = dsum_p.reshape(NC, N, D)
    h, ns, nd = _prep(x, dout_p, din_p)

    for W, b, relu_next in ((W1, b1, True), (W2, b2, True), (W3, b3, False)):
        parts = _spmm(h, srcr_s, dstr_s, zrows).reshape(NC, N, D)
        h = _layer(parts, ns, nd, W, b, relu_next)
    return h


# speedup vs baseline: 1.7731x; 1.0187x over previous
import functools

import jax
import jax.numpy as jnp
from jax import lax
from jax.experimental import pallas as pl
from jax.experimental.pallas import tpu as pltpu
from jax.experimental.pallas import tpu_sc as plsc

N = 10000
E = 320000
D = 128
NC = 2
NS = 16
NW = NC * NS
EPW = E // NW
CHUNK = 125
NCHUNK = EPW // CHUNK
CH_S = 50
NCH_S = EPW // CH_S
KB = 8
NBLK = NCH_S // KB
SLOTR = 2 * KB
MBUF = 6
NBUF = 5
AP = 624
ZW = 640
DW = 16

_mesh = plsc.VectorSubcoreMesh(core_axis_name="c", subcore_axis_name="s")


def _count_body(srcr, dstr, zrows, ones_hbm, dout_hbm, dsum_hbm,
                sidx_v, didx_v, ones_v, acc, ssem):
    cid = lax.axis_index("c")
    sid = lax.axis_index("s")
    wid = cid * NS + sid

    pltpu.sync_copy(ones_hbm, ones_v)
    base = sid * AP
    pltpu.sync_copy(zrows, acc.at[pl.ds(base, ZW)])
    pltpu.sync_copy(srcr.at[wid], sidx_v)
    pltpu.sync_copy(dstr.at[wid], didx_v)
    plsc.subcore_barrier()

    @pl.loop(0, NCHUNK, step=8)
    def _(j0):
        for t in range(8):
            pltpu.async_copy(ones_v, acc.at[sidx_v.at[j0 + t]], ssem, add=True)

        @pl.loop(0, 8)
        def _(_t):
            pltpu.make_async_copy(ones_v, acc.at[sidx_v.at[j0]], ssem).wait()

    plsc.subcore_barrier()
    pltpu.sync_copy(acc.at[pl.ds(base, ZW)],
                    dout_hbm.at[pl.ds(cid * N + base, ZW)])
    plsc.subcore_barrier()

    @pl.loop(0, NCHUNK, step=8)
    def _(j0):
        for t in range(8):
            pltpu.async_copy(ones_v, acc.at[didx_v.at[j0 + t]], ssem, add=True)

        @pl.loop(0, 8)
        def _(_t):
            pltpu.make_async_copy(ones_v, acc.at[didx_v.at[j0]], ssem).wait()

    plsc.subcore_barrier()
    pltpu.sync_copy(acc.at[pl.ds(base, ZW)],
                    dsum_hbm.at[pl.ds(cid * N + base, ZW)])


_count = pl.kernel(
    _count_body,
    out_type=[
        jax.ShapeDtypeStruct((NC * N, D), jnp.float32),
        jax.ShapeDtypeStruct((NC * N, D), jnp.float32),
    ],
    mesh=_mesh,
    scratch_types=[
        pltpu.VMEM((NCHUNK, CHUNK), jnp.int32),
        pltpu.VMEM((NCHUNK, CHUNK), jnp.int32),
        pltpu.VMEM((CHUNK, D), jnp.float32),
        pltpu.VMEM_SHARED((N, D), jnp.float32),
        pltpu.SemaphoreType.DMA,
    ],
)


def _spmm_body(h_hbm, srcrB, dstrB, zrows, out_hbm,
               sidx, didx, rows_v, acc, gsem, ssem, bsem):
    cid = lax.axis_index("c")
    sid = lax.axis_index("s")
    wid = cid * NS + sid

    base = sid * AP
    pltpu.sync_copy(zrows, acc.at[pl.ds(base, ZW)])
    pltpu.sync_copy(srcrB.at[wid, 0], sidx.at[pl.ds(0, KB)])
    pltpu.sync_copy(dstrB.at[wid, 0], didx.at[pl.ds(0, KB)])
    pltpu.async_copy(srcrB.at[wid, 1], sidx.at[pl.ds(KB, KB)], bsem.at[1])
    pltpu.async_copy(dstrB.at[wid, 1], didx.at[pl.ds(KB, KB)], bsem.at[1])
    plsc.subcore_barrier()

    for b in range(NBUF):
        pltpu.async_copy(h_hbm.at[sidx.at[b]], rows_v.at[b], gsem.at[b])

    @pl.loop(0, NCH_S)
    def _(j):
        b = lax.rem(j, MBUF)
        pltpu.make_async_copy(h_hbm.at[sidx.at[0]], rows_v.at[b],
                              gsem.at[b]).wait()
        pltpu.async_copy(rows_v.at[b], acc.at[didx.at[lax.rem(j, SLOTR)]],
                         ssem.at[b], add=True)
        jn = j + NBUF

        @pl.when(jn < NCH_S)
        def _():
            bn = lax.rem(jn, MBUF)

            @pl.when(jn >= MBUF)
            def _():
                pltpu.make_async_copy(rows_v.at[bn], acc.at[didx.at[0]],
                                      ssem.at[bn]).wait()

            @pl.when(lax.rem(jn, KB) == 0)
            def _():
                p = lax.rem(jn // KB, 2)
                pltpu.make_async_copy(srcrB.at[wid, 0], sidx.at[pl.ds(0, KB)],
                                      bsem.at[p]).wait()
                pltpu.make_async_copy(dstrB.at[wid, 0], didx.at[pl.ds(0, KB)],
                                      bsem.at[p]).wait()

            pltpu.async_copy(h_hbm.at[sidx.at[lax.rem(jn, SLOTR)]],
                             rows_v.at[bn], gsem.at[bn])

        blk_next = j // KB + 1

        @pl.when(jnp.logical_and(lax.rem(j, KB) == 2,
                                 jnp.logical_and(blk_next >= 2,
                                                 blk_next < NBLK)))
        def _():
            p = lax.rem(blk_next, 2)
            off = pl.multiple_of(p * KB, 8)
            pltpu.async_copy(srcrB.at[wid, blk_next],
                             sidx.at[pl.ds(off, KB)], bsem.at[p])
            pltpu.async_copy(dstrB.at[wid, blk_next],
                             didx.at[pl.ds(off, KB)], bsem.at[p])

    @pl.loop(0, MBUF)
    def _(b):
        pltpu.make_async_copy(rows_v.at[b], acc.at[didx.at[0]],
                              ssem.at[b]).wait()

    plsc.subcore_barrier()
    pltpu.sync_copy(acc.at[pl.ds(base, ZW)],
                    out_hbm.at[pl.ds(cid * N + base, ZW)])


_spmm = pl.kernel(
    _spmm_body,
    out_type=jax.ShapeDtypeStruct((NC * N, D), jnp.float32),
    mesh=_mesh,
    scratch_types=[
        pltpu.VMEM((SLOTR, CH_S), jnp.int32),
        pltpu.VMEM((SLOTR, CH_S), jnp.int32),
        pltpu.VMEM((MBUF, CH_S, D), jnp.float32),
        pltpu.VMEM_SHARED((N, D), jnp.float32),
        pltpu.SemaphoreType.DMA((MBUF,)),
        pltpu.SemaphoreType.DMA((MBUF,)),
        pltpu.SemaphoreType.DMA((2,)),
    ],
)


_BT = 1000


def _prep_body(x_ref, dop_ref, dip_ref, xs_ref, ns_ref, nd_ref):
    dout = dop_ref[0][:, :DW] + dop_ref[1][:, :DW]
    din = dip_ref[0][:, :DW] + dip_ref[1][:, :DW] - dout
    ns = lax.rsqrt(jnp.maximum(dout, 1.0))
    nd = lax.rsqrt(jnp.maximum(din, 1.0))
    ns_ref[...] = ns
    nd_ref[...] = nd
    xs_ref[...] = x_ref[...] * ns[:, 0:1]


def _layer_body(relu_next, p_ref, ns_ref, nd_ref, w_ref, b_ref, o_ref):
    agg = (p_ref[0] + p_ref[1]) * nd_ref[...][:, 0:1]
    h = jnp.dot(agg, w_ref[...], preferred_element_type=jnp.float32)
    h = h + b_ref[...]
    if relu_next:
        h = jnp.maximum(h, 0.0) * ns_ref[...][:, 0:1]
    o_ref[...] = h


def _prep(x, dout_p, din_p):
    grid = N // _BT
    return pl.pallas_call(
        _prep_body,
        grid=(grid,),
        in_specs=[
            pl.BlockSpec((_BT, D), lambda i: (i, 0)),
            pl.BlockSpec((NC, _BT, D), lambda i: (0, i, 0)),
            pl.BlockSpec((NC, _BT, D), lambda i: (0, i, 0)),
        ],
        out_specs=[
            pl.BlockSpec((_BT, D), lambda i: (i, 0)),
            pl.BlockSpec((_BT, DW), lambda i: (i, 0)),
            pl.BlockSpec((_BT, DW), lambda i: (i, 0)),
        ],
        out_shape=[
            jax.ShapeDtypeStruct((N, D), jnp.float32),
            jax.ShapeDtypeStruct((N, DW), jnp.float32),
            jax.ShapeDtypeStruct((N, DW), jnp.float32),
        ],
    )(x, dout_p, din_p)


def _layer(parts, ns, nd, W, b, relu_next):
    grid = N // _BT
    return pl.pallas_call(
        functools.partial(_layer_body, relu_next),
        grid=(grid,),
        in_specs=[
            pl.BlockSpec((NC, _BT, D), lambda i: (0, i, 0)),
            pl.BlockSpec((_BT, DW), lambda i: (i, 0)),
            pl.BlockSpec((_BT, DW), lambda i: (i, 0)),
            pl.BlockSpec((D, D), lambda i: (0, 0)),
            pl.BlockSpec((1, D), lambda i: (0, 0)),
        ],
        out_specs=pl.BlockSpec((_BT, D), lambda i: (i, 0)),
        out_shape=jax.ShapeDtypeStruct((N, D), jnp.float32),
    )(parts, ns, nd, W, b.reshape(1, D))


def kernel(x, edge_index, W1, b1, W2, b2, W3, b3):
    ei = edge_index.astype(jnp.int32)
    srcr_c = ei[0].reshape(NW, NCHUNK, CHUNK)
    dstr_c = ei[1].reshape(NW, NCHUNK, CHUNK)
    srcr_s = ei[0].reshape(NW, NBLK, KB, CH_S)
    dstr_s = ei[1].reshape(NW, NBLK, KB, CH_S)
    zrows = jnp.zeros((ZW, D), jnp.float32)
    ones = jnp.ones((CHUNK, D), jnp.float32)

    dout_p, dsum_p = _count(srcr_c, dstr_c, zrows, ones)
    dout_p = dout_p.reshape(NC, N, D)
    din_p = dsum_p.reshape(NC, N, D)
    h, ns, nd = _prep(x, dout_p, din_p)

    for W, b, relu_next in ((W1, b1, True), (W2, b2, True), (W3, b3, False)):
        parts = _spmm(h, srcr_s, dstr_s, zrows).reshape(NC, N, D)
        h = _layer(parts, ns, nd, W, b, relu_next)
    return h
